# baseline (speedup 1.0000x reference)
import jax
import jax.numpy as jnp
from jax.experimental import pallas as pl
from jax.experimental.pallas import tpu as pltpu


def _router_kernel(x_ref, wc_ref, bst_ref, p_ref, rb_ref, cm_ref,
                   logits_ref, weights_ref, pooled_ref):
    s = pl.program_id(1)
    ns = pl.num_programs(1)
    x = x_ref[0]

    out = jnp.dot(x, wc_ref[...], preferred_element_type=jnp.float32)

    out2 = jnp.dot(out, p_ref[...], preferred_element_type=jnp.float32)
    out2 = out2 + rb_ref[...]
    logits_ref[0] = out2[:, :8]
    m = jnp.max(out2, axis=-1, keepdims=True)
    e = jnp.exp(out2 - m)
    w = e / jnp.sum(e, axis=-1, keepdims=True)
    weights_ref[0] = w[:, :8]

    ts = jnp.tanh(out + bst_ref[...]) * cm_ref[...]
    part = jnp.sum(ts, axis=0, keepdims=True)

    @pl.when(s == 0)
    def _init():
        pooled_ref[0] = jnp.zeros_like(pooled_ref[0])

    pooled_ref[0] += part[:, :64]

    @pl.when(s == ns - 1)
    def _finish():
        pooled_ref[0] = pooled_ref[0] * (1.0 / (x_ref.shape[1] * ns))


def kernel(hidden, W_state, b_state, W_route, b_route):
    B, S, H = hidden.shape
    SD = W_state.shape[1]
    E = W_route.shape[1]
    BS = 4096
    ns = S // BS

    W_comb = jnp.concatenate(
        [W_state, W_route, jnp.zeros((H, 128 - SD - E), jnp.float32)], axis=1)
    P = jnp.zeros((128, 128), jnp.float32)
    P = P.at[SD + jnp.arange(E), jnp.arange(E)].set(1.0)
    rowbias = jnp.full((128,), -1e30, jnp.float32)
    rowbias = rowbias.at[:E].set(b_route).reshape(1, 128)
    bst_full = jnp.zeros((128,), jnp.float32).at[:SD].set(b_state).reshape(1, 128)
    colmask = (jnp.arange(128) < SD).astype(jnp.float32).reshape(1, 128)

    grid = (B, ns)
    out_shape = (
        jax.ShapeDtypeStruct((B, S, E), jnp.float32),
        jax.ShapeDtypeStruct((B, S, E), jnp.float32),
        jax.ShapeDtypeStruct((B, 1, SD), jnp.float32),
    )
    logits, weights, pooled = pl.pallas_call(
        _router_kernel,
        grid=grid,
        in_specs=[
            pl.BlockSpec((1, BS, H), lambda b, s: (b, s, 0)),
            pl.BlockSpec((H, 128), lambda b, s: (0, 0)),
            pl.BlockSpec((1, 128), lambda b, s: (0, 0)),
            pl.BlockSpec((128, 128), lambda b, s: (0, 0)),
            pl.BlockSpec((1, 128), lambda b, s: (0, 0)),
            pl.BlockSpec((1, 128), lambda b, s: (0, 0)),
        ],
        out_specs=(
            pl.BlockSpec((1, BS, E), lambda b, s: (b, s, 0)),
            pl.BlockSpec((1, BS, E), lambda b, s: (b, s, 0)),
            pl.BlockSpec((1, 1, SD), lambda b, s: (b, 0, 0)),
        ),
        out_shape=out_shape,
        compiler_params=pltpu.CompilerParams(
            dimension_semantics=("parallel", "arbitrary"),
        ),
    )(hidden, W_comb, bst_full, P, rowbias, colmask)
    return (logits, weights, pooled)
